# baseline (speedup 1.0000x reference)
import functools

import jax
import jax.numpy as jnp
from jax import lax
from jax.experimental import pallas as pl
from jax.experimental.pallas import tpu as pltpu
from jax.experimental.pallas import tpu_sc as plsc

D_MODEL = 2048
FFN = 2048
E = 8
T = 2048
BT = 1024
NB = (2 * T) // BT + E
L = NB * BT
NF = 8
BF = FFN // NF

_SC_NC = 2
_SC_NS = 16
_SC_NW = _SC_NC * _SC_NS


def _router_kernel(x_ref, gw_ref, wts_ref, pos_ref, be_ref, bv_ref):
    logits = lax.dot_general(
        x_ref[...], gw_ref[...], (((1,), (1,)), ((), ())),
        preferred_element_type=jnp.float32)
    m = jnp.max(logits, axis=1, keepdims=True)
    p = jnp.exp(logits - m)
    p = p / jnp.sum(p, axis=1, keepdims=True)
    lane = lax.broadcasted_iota(jnp.int32, p.shape, 1)
    m1 = jnp.max(p, axis=1, keepdims=True)
    i1 = jnp.min(jnp.where(p == m1, lane, E), axis=1, keepdims=True)
    p2 = jnp.where(lane == i1, -1.0, p)
    m2 = jnp.max(p2, axis=1, keepdims=True)
    i2 = jnp.min(jnp.where(p2 == m2, lane, E), axis=1, keepdims=True)
    s = m1 + m2
    wts_ref[...] = jnp.concatenate([m1 / s, m2 / s], axis=1)

    oh0 = (lane == i1).astype(jnp.float32)
    oh1 = (lane == i2).astype(jnp.float32)
    ss = oh0 + oh1
    c = ss
    sh = 1
    while sh < T:
        c = c + jnp.concatenate(
            [jnp.zeros((sh, E), jnp.float32), c[:T - sh]], axis=0)
        sh *= 2
    counts = c[T - 1:T, :]
    c = c - ss
    nblk = jnp.floor((counts + (BT - 1)) * (1.0 / BT))
    ends = nblk
    sh = 1
    while sh < E:
        ends = ends + jnp.concatenate(
            [jnp.zeros((1, sh), jnp.float32), ends[:, :E - sh]], axis=1)
        sh *= 2
    row_off = (ends - nblk) * BT
    rank0 = jnp.sum(c * oh0, axis=1, keepdims=True)
    rank1 = jnp.sum((c + oh0) * oh1, axis=1, keepdims=True)
    off0 = jnp.sum(row_off * oh0, axis=1, keepdims=True)
    off1 = jnp.sum(row_off * oh1, axis=1, keepdims=True)
    pos_ref[...] = jnp.concatenate(
        [off0 + rank0, off1 + rank1], axis=1).astype(jnp.int32)

    b_iota = lax.broadcasted_iota(jnp.int32, (1, NB), 1).astype(jnp.float32)
    be = jnp.zeros((1, NB), jnp.int32)
    for e in range(E - 1):
        be = be + (ends[0, e] <= b_iota).astype(jnp.int32)
    be_ref[...] = be
    bv_ref[...] = (b_iota < ends[0, E - 1]).astype(jnp.int32)


def _route(x, gate_w):
    return pl.pallas_call(
        _router_kernel,
        out_shape=(jax.ShapeDtypeStruct((T, 2), jnp.float32),
                   jax.ShapeDtypeStruct((T, 2), jnp.int32),
                   jax.ShapeDtypeStruct((1, NB), jnp.int32),
                   jax.ShapeDtypeStruct((1, NB), jnp.int32)),
    )(x, gate_w)


def _sc_mesh():
    return plsc.VectorSubcoreMesh(core_axis_name="c", subcore_axis_name="s")


def _scatter_rows(src, idx_e, idx_o, n_out, chunk):
    n_rows, d = src.shape
    r_per_w = n_rows // _SC_NW
    n_ch = r_per_w // chunk

    @functools.partial(
        pl.kernel, mesh=_sc_mesh(),
        out_type=jax.ShapeDtypeStruct((n_out, d), src.dtype),
        scratch_types=[pltpu.VMEM((n_ch, chunk), jnp.int32),
                       pltpu.VMEM((n_ch, chunk), jnp.int32),
                       pltpu.VMEM((chunk, d), src.dtype),
                       pltpu.SemaphoreType.DMA,
                       pltpu.SemaphoreType.DMA],
    )
    def k(src_hbm, ie_hbm, io_hbm, out_hbm, ie_v, io_v, rows_v, sem_e, sem_o):
        wid = lax.axis_index("s") * _SC_NC + lax.axis_index("c")
        base = wid * r_per_w
        pltpu.sync_copy(ie_hbm.at[wid], ie_v)
        pltpu.sync_copy(io_hbm.at[wid], io_v)

        @pl.loop(0, n_ch)
        def _(c):
            pltpu.sync_copy(src_hbm.at[pl.ds(base + c * chunk, chunk)], rows_v)
            a = pltpu.async_copy(rows_v, out_hbm.at[ie_v.at[c]], sem_e)
            b = pltpu.async_copy(rows_v, out_hbm.at[io_v.at[c]], sem_o)
            a.wait()
            b.wait()

    return k(src, idx_e, idx_o)


def _gather_rows(table, idx, chunk):
    n_rows, d = table.shape
    b = idx.shape[0]
    b_per_w = b // _SC_NW
    n_ch = b_per_w // chunk

    @functools.partial(
        pl.kernel, mesh=_sc_mesh(),
        out_type=jax.ShapeDtypeStruct((b, d), table.dtype),
        scratch_types=[pltpu.VMEM((b_per_w,), jnp.int32),
                       pltpu.VMEM((chunk, d), table.dtype),
                       pltpu.SemaphoreType.DMA],
    )
    def k(table_hbm, idx_hbm, out_hbm, idx_v, rows_v, sem):
        wid = lax.axis_index("s") * _SC_NC + lax.axis_index("c")
        base = wid * b_per_w
        pltpu.sync_copy(idx_hbm.at[pl.ds(base, b_per_w)], idx_v)

        @pl.loop(0, n_ch)
        def _(c):
            pltpu.async_copy(
                table_hbm.at[idx_v.at[pl.ds(c * chunk, chunk)]],
                rows_v, sem).wait()
            pltpu.sync_copy(rows_v, out_hbm.at[pl.ds(base + c * chunk, chunk)])

    return k(table, idx)


def _glu_kernel(be_ref, bv_ref, xs_ref, w1_ref, v1_ref, w2_ref, ys_ref):
    b = pl.program_id(0)
    f = pl.program_id(1)

    @pl.when(bv_ref[b] > 0)
    def _():
        xb = xs_ref[...].astype(jnp.bfloat16)
        w1t = w1_ref[0].astype(jnp.bfloat16)
        v1t = v1_ref[0].astype(jnp.bfloat16)
        w2t = w2_ref[0].astype(jnp.bfloat16)
        a = lax.dot_general(xb, w1t, (((1,), (1,)), ((), ())),
                            preferred_element_type=jnp.float32)
        bb = lax.dot_general(xb, v1t, (((1,), (1,)), ((), ())),
                             preferred_element_type=jnp.float32)
        h = (a * jax.nn.sigmoid(a) * bb).astype(jnp.bfloat16)
        yf = lax.dot_general(h, w2t, (((1,), (0,)), ((), ())),
                             preferred_element_type=jnp.float32)

        @pl.when(f == 0)
        def _():
            ys_ref[...] = yf

        @pl.when(f != 0)
        def _():
            ys_ref[...] += yf


def _grouped_glu(block_expert, block_valid, xs, w1r, v1r, w2r):
    grid_spec = pltpu.PrefetchScalarGridSpec(
        num_scalar_prefetch=2,
        grid=(NB, NF),
        in_specs=[
            pl.BlockSpec((BT, D_MODEL),
                         lambda b, f, be, bv: (jnp.where(bv[b] > 0, b, 0), 0)),
            pl.BlockSpec((1, BF, D_MODEL),
                         lambda b, f, be, bv:
                         (be[b], jnp.where(bv[b] > 0, f, NF - 1), 0)),
            pl.BlockSpec((1, BF, D_MODEL),
                         lambda b, f, be, bv:
                         (be[b], jnp.where(bv[b] > 0, f, NF - 1), 0)),
            pl.BlockSpec((1, BF, D_MODEL),
                         lambda b, f, be, bv:
                         (be[b], jnp.where(bv[b] > 0, f, NF - 1), 0)),
        ],
        out_specs=pl.BlockSpec(
            (BT, D_MODEL),
            lambda b, f, be, bv: (jnp.where(bv[b] > 0, b, NB - 1), 0)),
    )
    return pl.pallas_call(
        _glu_kernel,
        grid_spec=grid_spec,
        out_shape=jax.ShapeDtypeStruct((L, D_MODEL), jnp.float32),
    )(block_expert, block_valid, xs, w1r, v1r, w2r)


def _combine_kernel(g2_ref, wts_ref, out_ref):
    w0 = wts_ref[:, 0:1]
    w1 = wts_ref[:, 1:2]
    out_ref[...] = (w0 * g2_ref[:, :D_MODEL] + w1 * g2_ref[:, D_MODEL:])


def _combine(g2, wts):
    bt = 256
    return pl.pallas_call(
        _combine_kernel,
        grid=(T // bt,),
        in_specs=[
            pl.BlockSpec((bt, 2 * D_MODEL), lambda i: (i, 0)),
            pl.BlockSpec((bt, 2), lambda i: (i, 0)),
        ],
        out_specs=pl.BlockSpec((bt, D_MODEL), lambda i: (i, 0)),
        out_shape=jax.ShapeDtypeStruct((T, D_MODEL), jnp.float32),
    )(g2, wts)


def kernel(x, gate_w, w1, v1, w2):
    wts, pos, be, bv = _route(x, gate_w)

    n_ch = (T // _SC_NW) // 16
    idx_e = pos[:, 0].reshape(_SC_NW, n_ch, 16)
    idx_o = pos[:, 1].reshape(_SC_NW, n_ch, 16)

    xs = _scatter_rows(x, idx_e, idx_o, L, 16)

    w1r = w1.reshape(E, FFN, D_MODEL)
    v1r = v1.reshape(E, FFN, D_MODEL)
    w2r = w2.reshape(E, FFN, D_MODEL)
    ys = _grouped_glu(be.reshape(NB), bv.reshape(NB), xs, w1r, v1r, w2r)

    g = _gather_rows(ys, pos.reshape(2 * T), 16)
    g2 = g.reshape(T, 2 * D_MODEL)
    return _combine(g2, wts)


# speedup vs baseline: 1.1022x; 1.1022x over previous
import functools

import jax
import jax.numpy as jnp
from jax import lax
from jax.experimental import pallas as pl
from jax.experimental.pallas import tpu as pltpu
from jax.experimental.pallas import tpu_sc as plsc

D_MODEL = 2048
FFN = 2048
E = 8
T = 2048
BT = 512
NB = (2 * T) // BT + E
L = NB * BT
NF = 4
BF = FFN // NF

_SC_NC = 2
_SC_NS = 16
_SC_NW = _SC_NC * _SC_NS


def _router_kernel(x_ref, gw_ref, wts_ref, pos_ref, be_ref, bv_ref):
    logits = lax.dot_general(
        x_ref[...], gw_ref[...], (((1,), (1,)), ((), ())),
        preferred_element_type=jnp.float32)
    m = jnp.max(logits, axis=1, keepdims=True)
    p = jnp.exp(logits - m)
    p = p / jnp.sum(p, axis=1, keepdims=True)
    lane = lax.broadcasted_iota(jnp.int32, p.shape, 1)
    m1 = jnp.max(p, axis=1, keepdims=True)
    i1 = jnp.min(jnp.where(p == m1, lane, E), axis=1, keepdims=True)
    p2 = jnp.where(lane == i1, -1.0, p)
    m2 = jnp.max(p2, axis=1, keepdims=True)
    i2 = jnp.min(jnp.where(p2 == m2, lane, E), axis=1, keepdims=True)
    s = m1 + m2
    wts_ref[...] = jnp.concatenate([m1 / s, m2 / s], axis=1)

    oh0 = (lane == i1).astype(jnp.float32)
    oh1 = (lane == i2).astype(jnp.float32)
    ss = oh0 + oh1
    c = ss
    sh = 1
    while sh < T:
        c = c + jnp.concatenate(
            [jnp.zeros((sh, E), jnp.float32), c[:T - sh]], axis=0)
        sh *= 2
    counts = c[T - 1:T, :]
    c = c - ss
    nblk = jnp.floor((counts + (BT - 1)) * (1.0 / BT))
    ends = nblk
    sh = 1
    while sh < E:
        ends = ends + jnp.concatenate(
            [jnp.zeros((1, sh), jnp.float32), ends[:, :E - sh]], axis=1)
        sh *= 2
    row_off = (ends - nblk) * BT
    rank0 = jnp.sum(c * oh0, axis=1, keepdims=True)
    rank1 = jnp.sum((c + oh0) * oh1, axis=1, keepdims=True)
    off0 = jnp.sum(row_off * oh0, axis=1, keepdims=True)
    off1 = jnp.sum(row_off * oh1, axis=1, keepdims=True)
    pos_ref[...] = jnp.concatenate(
        [off0 + rank0, off1 + rank1], axis=1).astype(jnp.int32)

    b_iota = lax.broadcasted_iota(jnp.int32, (1, NB), 1).astype(jnp.float32)
    be = jnp.zeros((1, NB), jnp.int32)
    for e in range(E - 1):
        be = be + (ends[0, e] <= b_iota).astype(jnp.int32)
    be_ref[...] = be
    bv_ref[...] = (b_iota < ends[0, E - 1]).astype(jnp.int32)


def _route(x, gate_w):
    return pl.pallas_call(
        _router_kernel,
        out_shape=(jax.ShapeDtypeStruct((T, 2), jnp.float32),
                   jax.ShapeDtypeStruct((T, 2), jnp.int32),
                   jax.ShapeDtypeStruct((1, NB), jnp.int32),
                   jax.ShapeDtypeStruct((1, NB), jnp.int32)),
    )(x, gate_w)


def _sc_mesh():
    return plsc.VectorSubcoreMesh(core_axis_name="c", subcore_axis_name="s")


def _scatter_rows(src, idx_e, idx_o, n_out, chunk):
    n_rows, d = src.shape
    r_per_w = n_rows // _SC_NW
    n_ch = r_per_w // chunk

    @functools.partial(
        pl.kernel, mesh=_sc_mesh(),
        out_type=jax.ShapeDtypeStruct((n_out, d), src.dtype),
        scratch_types=[pltpu.VMEM((n_ch, chunk), jnp.int32),
                       pltpu.VMEM((n_ch, chunk), jnp.int32),
                       pltpu.VMEM((chunk, d), src.dtype),
                       pltpu.VMEM((chunk, d), src.dtype),
                       pltpu.SemaphoreType.DMA,
                       pltpu.SemaphoreType.DMA,
                       pltpu.SemaphoreType.DMA,
                       pltpu.SemaphoreType.DMA,
                       pltpu.SemaphoreType.DMA,
                       pltpu.SemaphoreType.DMA],
    )
    def k(src_hbm, ie_hbm, io_hbm, out_hbm, ie_v, io_v, rows_v0, rows_v1,
          semr0, semr1, sem_e0, sem_o0, sem_e1, sem_o1):
        wid = lax.axis_index("s") * _SC_NC + lax.axis_index("c")
        base = wid * r_per_w
        pltpu.sync_copy(ie_hbm.at[wid], ie_v)
        pltpu.sync_copy(io_hbm.at[wid], io_v)

        @pl.loop(0, n_ch, step=2)
        def _(c):
            r0 = pltpu.async_copy(
                src_hbm.at[pl.ds(base + c * chunk, chunk)], rows_v0, semr0)
            r1 = pltpu.async_copy(
                src_hbm.at[pl.ds(base + (c + 1) * chunk, chunk)], rows_v1,
                semr1)
            r0.wait()
            a0 = pltpu.async_copy(rows_v0, out_hbm.at[ie_v.at[c]], sem_e0)
            b0 = pltpu.async_copy(rows_v0, out_hbm.at[io_v.at[c]], sem_o0)
            r1.wait()
            a1 = pltpu.async_copy(rows_v1, out_hbm.at[ie_v.at[c + 1]], sem_e1)
            b1 = pltpu.async_copy(rows_v1, out_hbm.at[io_v.at[c + 1]], sem_o1)
            a0.wait()
            b0.wait()
            a1.wait()
            b1.wait()

    return k(src, idx_e, idx_o)


def _gather_rows(table, idx, chunk):
    n_rows, d = table.shape
    b = idx.shape[0]
    b_per_w = b // _SC_NW
    n_ch = b_per_w // chunk

    @functools.partial(
        pl.kernel, mesh=_sc_mesh(),
        out_type=jax.ShapeDtypeStruct((b, d), table.dtype),
        scratch_types=[pltpu.VMEM((b_per_w,), jnp.int32),
                       pltpu.VMEM((chunk, d), table.dtype),
                       pltpu.VMEM((chunk, d), table.dtype),
                       pltpu.SemaphoreType.DMA,
                       pltpu.SemaphoreType.DMA,
                       pltpu.SemaphoreType.DMA,
                       pltpu.SemaphoreType.DMA],
    )
    def k(table_hbm, idx_hbm, out_hbm, idx_v, rows_v0, rows_v1,
          semg0, semg1, semw0, semw1):
        wid = lax.axis_index("s") * _SC_NC + lax.axis_index("c")
        base = wid * b_per_w
        pltpu.sync_copy(idx_hbm.at[pl.ds(base, b_per_w)], idx_v)

        @pl.loop(0, n_ch, step=2)
        def _(c):
            g0 = pltpu.async_copy(
                table_hbm.at[idx_v.at[pl.ds(c * chunk, chunk)]],
                rows_v0, semg0)
            g1 = pltpu.async_copy(
                table_hbm.at[idx_v.at[pl.ds((c + 1) * chunk, chunk)]],
                rows_v1, semg1)
            g0.wait()
            w0 = pltpu.async_copy(
                rows_v0, out_hbm.at[pl.ds(base + c * chunk, chunk)], semw0)
            g1.wait()
            w1 = pltpu.async_copy(
                rows_v1, out_hbm.at[pl.ds(base + (c + 1) * chunk, chunk)],
                semw1)
            w0.wait()
            w1.wait()

    return k(table, idx)


def _glu_kernel(be_ref, bv_ref, xs_ref, w1_ref, v1_ref, w2_ref, ys_ref):
    b = pl.program_id(0)
    f = pl.program_id(1)

    @pl.when(bv_ref[b] > 0)
    def _():
        xb = xs_ref[...].astype(jnp.bfloat16)
        w1t = w1_ref[0].astype(jnp.bfloat16)
        v1t = v1_ref[0].astype(jnp.bfloat16)
        w2t = w2_ref[0].astype(jnp.bfloat16)
        a = lax.dot_general(xb, w1t, (((1,), (1,)), ((), ())),
                            preferred_element_type=jnp.float32)
        bb = lax.dot_general(xb, v1t, (((1,), (1,)), ((), ())),
                             preferred_element_type=jnp.float32)
        h = (a * jax.nn.sigmoid(a) * bb).astype(jnp.bfloat16)
        yf = lax.dot_general(h, w2t, (((1,), (0,)), ((), ())),
                             preferred_element_type=jnp.float32)

        @pl.when(f == 0)
        def _():
            ys_ref[...] = yf

        @pl.when(f != 0)
        def _():
            ys_ref[...] += yf


def _grouped_glu(block_expert, block_valid, xs, w1r, v1r, w2r):
    grid_spec = pltpu.PrefetchScalarGridSpec(
        num_scalar_prefetch=2,
        grid=(NB, NF),
        in_specs=[
            pl.BlockSpec((BT, D_MODEL),
                         lambda b, f, be, bv: (jnp.where(bv[b] > 0, b, 0), 0)),
            pl.BlockSpec((1, BF, D_MODEL),
                         lambda b, f, be, bv:
                         (be[b], jnp.where(bv[b] > 0, f, NF - 1), 0)),
            pl.BlockSpec((1, BF, D_MODEL),
                         lambda b, f, be, bv:
                         (be[b], jnp.where(bv[b] > 0, f, NF - 1), 0)),
            pl.BlockSpec((1, BF, D_MODEL),
                         lambda b, f, be, bv:
                         (be[b], jnp.where(bv[b] > 0, f, NF - 1), 0)),
        ],
        out_specs=pl.BlockSpec(
            (BT, D_MODEL),
            lambda b, f, be, bv: (jnp.where(bv[b] > 0, b, NB - 1), 0)),
    )
    return pl.pallas_call(
        _glu_kernel,
        grid_spec=grid_spec,
        out_shape=jax.ShapeDtypeStruct((L, D_MODEL), jnp.float32),
    )(block_expert, block_valid, xs, w1r, v1r, w2r)


def _combine_kernel(g2_ref, wts_ref, out_ref):
    w0 = wts_ref[:, 0:1]
    w1 = wts_ref[:, 1:2]
    out_ref[...] = (w0 * g2_ref[:, :D_MODEL] + w1 * g2_ref[:, D_MODEL:])


def _combine(g2, wts):
    bt = 256
    return pl.pallas_call(
        _combine_kernel,
        grid=(T // bt,),
        in_specs=[
            pl.BlockSpec((bt, 2 * D_MODEL), lambda i: (i, 0)),
            pl.BlockSpec((bt, 2), lambda i: (i, 0)),
        ],
        out_specs=pl.BlockSpec((bt, D_MODEL), lambda i: (i, 0)),
        out_shape=jax.ShapeDtypeStruct((T, D_MODEL), jnp.float32),
    )(g2, wts)


def kernel(x, gate_w, w1, v1, w2):
    wts, pos, be, bv = _route(x, gate_w)

    n_ch = (T // _SC_NW) // 16
    idx_e = pos[:, 0].reshape(_SC_NW, n_ch, 16)
    idx_o = pos[:, 1].reshape(_SC_NW, n_ch, 16)

    xs = _scatter_rows(x, idx_e, idx_o, L, 16)

    w1r = w1.reshape(E, FFN, D_MODEL)
    v1r = v1.reshape(E, FFN, D_MODEL)
    w2r = w2.reshape(E, FFN, D_MODEL)
    ys = _grouped_glu(be.reshape(NB), bv.reshape(NB), xs, w1r, v1r, w2r)

    g = _gather_rows(ys, pos.reshape(2 * T), 16)
    g2 = g.reshape(T, 2 * D_MODEL)
    return _combine(g2, wts)
